# Initial kernel scaffold; baseline (speedup 1.0000x reference)
import jax
import jax.numpy as jnp
from jax.experimental import pallas as pl


def kernel(x_vecs, device_weights, device_bias, category_weights, category_bias):
    raise NotImplementedError("write your pallas kernel here")


# speedup vs baseline: 3.4467x; 3.4467x over previous
import functools

import jax
import jax.numpy as jnp
from jax import lax
from jax.experimental import pallas as pl
from jax.experimental.pallas import tpu as pltpu
from jax.experimental.pallas import tpu_sc as plsc

B, R, H, W = 128, 34, 512, 512
NUM_CATS = 5
PIX = H * W
IMG = 2 * PIX
TOTAL = B * IMG
NW = 32
B_PER_W = B // NW
ZCHUNK = 65536
NZ = B_PER_W * IMG // ZCHUNK
RPAD = 48

_mesh = plsc.VectorSubcoreMesh(core_axis_name="c", subcore_axis_name="s")


@functools.partial(
    pl.kernel,
    mesh=_mesh,
    compiler_params=pltpu.CompilerParams(needs_layout_passes=False),
    out_type=jax.ShapeDtypeStruct((TOTAL,), jnp.float32),
    scratch_types=[
        pltpu.VMEM((ZCHUNK,), jnp.float32),
        pltpu.VMEM((B_PER_W * R * 4,), jnp.float32),
        pltpu.VMEM((R,), jnp.float32),
        pltpu.VMEM((R,), jnp.float32),
        pltpu.VMEM((NUM_CATS,), jnp.float32),
        pltpu.VMEM((NUM_CATS,), jnp.float32),
        pltpu.VMEM((B_PER_W, 2 * RPAD), jnp.int32),
        pltpu.VMEM((B_PER_W, 2 * RPAD), jnp.float32),
        pltpu.SemaphoreType.DMA,
        pltpu.SemaphoreType.DMA,
    ],
)
def _vec2im_sc(x_hbm, dw_hbm, db_hbm, cw_hbm, cb_hbm, out_hbm,
               zbuf, xbuf, dwb, dbb, cwb, cbb, idxb, valb, zsem, ssem):
    wid = lax.axis_index("s") * 2 + lax.axis_index("c")

    def _zbody(i, carry):
        base = i * 64
        z16 = jnp.zeros((16,), jnp.float32)
        zbuf[pl.ds(base, 16)] = z16
        zbuf[pl.ds(base + 16, 16)] = z16
        zbuf[pl.ds(base + 32, 16)] = z16
        zbuf[pl.ds(base + 48, 16)] = z16
        return carry

    lax.fori_loop(0, ZCHUNK // 64, _zbody, 0)

    region = wid * (B_PER_W * IMG)
    zcopies = [
        pltpu.async_copy(zbuf, out_hbm.at[pl.ds(region + k * ZCHUNK, ZCHUNK)],
                         zsem)
        for k in range(NZ)
    ]

    pltpu.sync_copy(x_hbm.at[pl.ds(wid * (B_PER_W * R * 4), B_PER_W * R * 4)],
                    xbuf)
    pltpu.sync_copy(dw_hbm, dwb)
    pltpu.sync_copy(db_hbm, dbb)
    pltpu.sync_copy(cw_hbm, cwb)
    pltpu.sync_copy(cb_hbm, cbb)

    lane = lax.iota(jnp.int32, 16)
    for bl in range(B_PER_W):
        gb = wid * B_PER_W + bl
        for c in range(RPAD // 16):
            rc = jnp.minimum(lane + c * 16, R - 1)
            base = bl * (R * 4) + rc * 4
            p = plsc.load_gather(xbuf, [base])
            xf = plsc.load_gather(xbuf, [base + 1])
            yf = plsc.load_gather(xbuf, [base + 2])
            cf = plsc.load_gather(xbuf, [base + 3])
            dw = plsc.load_gather(dwb, [rc])
            db = plsc.load_gather(dbb, [rc])
            cat = cf.astype(jnp.int32)
            cw = plsc.load_gather(cwb, [cat])
            cb = plsc.load_gather(cbb, [cat])
            ind = jnp.where(p != 0.0, 1.0, 0.0)
            v0 = (p * dw + ind * db) * cw + ind * cb
            xi = xf.astype(jnp.int32)
            yi = yf.astype(jnp.int32)
            pix = gb * IMG + yi * W + xi
            idxb[bl, pl.ds(c * 16, 16)] = pix
            idxb[bl, pl.ds(RPAD + c * 16, 16)] = pix + PIX
            valb[bl, pl.ds(c * 16, 16)] = v0
            valb[bl, pl.ds(RPAD + c * 16, 16)] = p

    for cp in zcopies:
        cp.wait()
    scopies = [
        pltpu.async_copy(valb.at[bl], out_hbm.at[idxb.at[bl]], ssem)
        for bl in range(B_PER_W)
    ]
    for cp in scopies:
        cp.wait()


def kernel(x_vecs, device_weights, device_bias, category_weights,
           category_bias):
    out = _vec2im_sc(x_vecs.reshape(-1), device_weights, device_bias,
                     category_weights, category_bias)
    return out.reshape(B, 2, H, W)


# speedup vs baseline: 12.4654x; 3.6166x over previous
import functools

import jax
import jax.numpy as jnp
from jax import lax
from jax.experimental import pallas as pl
from jax.experimental.pallas import tpu as pltpu
from jax.experimental.pallas import tpu_sc as plsc

B, R, H, W = 128, 34, 512, 512
NUM_CATS = 5
NW = 32
B_PER_W = B // NW
CROWS = 64
NCHUNK = H // CROWS
NCOMBO = B_PER_W * 2 * NCHUNK
RPAD = 48

_mesh = plsc.VectorSubcoreMesh(core_axis_name="c", subcore_axis_name="s")


@functools.partial(
    pl.kernel,
    mesh=_mesh,
    compiler_params=pltpu.CompilerParams(needs_layout_passes=False),
    out_type=jax.ShapeDtypeStruct((B, 2, H, W), jnp.float32),
    scratch_types=[
        pltpu.VMEM((CROWS, W), jnp.float32),
        pltpu.VMEM((CROWS, W), jnp.float32),
        pltpu.VMEM((B_PER_W * R * 4,), jnp.float32),
        pltpu.VMEM((R,), jnp.float32),
        pltpu.VMEM((R,), jnp.float32),
        pltpu.VMEM((NUM_CATS,), jnp.float32),
        pltpu.VMEM((NUM_CATS,), jnp.float32),
        pltpu.VMEM((B_PER_W, RPAD), jnp.int32),
        pltpu.VMEM((B_PER_W, RPAD), jnp.int32),
        pltpu.VMEM((2 * B_PER_W, RPAD), jnp.float32),
        pltpu.SemaphoreType.DMA,
        pltpu.SemaphoreType.DMA,
    ],
)
def _vec2im_sc(x_hbm, dw_hbm, db_hbm, cw_hbm, cb_hbm, out_hbm,
               buf0, buf1, xvb, dwb, dbb, cwb, cbb, xib, yib, vab,
               sem0, sem1):
    wid = lax.axis_index("s") * 2 + lax.axis_index("c")
    bufs = (buf0, buf1)
    sems = (sem0, sem1)

    def _zrow(r, carry):
        z16 = jnp.zeros((16,), jnp.float32)
        for j in range(W // 16):
            buf0[r, pl.ds(j * 16, 16)] = z16
            buf1[r, pl.ds(j * 16, 16)] = z16
        return carry

    lax.fori_loop(0, CROWS, _zrow, 0)

    pltpu.sync_copy(x_hbm.at[pl.ds(wid * (B_PER_W * R * 4), B_PER_W * R * 4)],
                    xvb)
    pltpu.sync_copy(dw_hbm, dwb)
    pltpu.sync_copy(db_hbm, dbb)
    pltpu.sync_copy(cw_hbm, cwb)
    pltpu.sync_copy(cb_hbm, cbb)

    lane = lax.iota(jnp.int32, 16)
    for bl in range(B_PER_W):
        for c in range(RPAD // 16):
            rc = jnp.minimum(lane + c * 16, R - 1)
            base = bl * (R * 4) + rc * 4
            p = plsc.load_gather(xvb, [base])
            xf = plsc.load_gather(xvb, [base + 1])
            yf = plsc.load_gather(xvb, [base + 2])
            cf = plsc.load_gather(xvb, [base + 3])
            dw = plsc.load_gather(dwb, [rc])
            db = plsc.load_gather(dbb, [rc])
            cat = cf.astype(jnp.int32)
            cw = plsc.load_gather(cwb, [cat])
            cb = plsc.load_gather(cbb, [cat])
            ind = jnp.where(p != 0.0, 1.0, 0.0)
            v0 = (p * dw + ind * db) * cw + ind * cb
            xib[bl, pl.ds(c * 16, 16)] = xf.astype(jnp.int32)
            yib[bl, pl.ds(c * 16, 16)] = yf.astype(jnp.int32)
            vab[2 * bl, pl.ds(c * 16, 16)] = v0
            vab[2 * bl + 1, pl.ds(c * 16, 16)] = p

    def _combo(c):
        return c // (2 * NCHUNK), (c // NCHUNK) % 2, c % NCHUNK

    def _scatter(buf, bl, chunk, values):
        for cc in range(RPAD // 16):
            xi = xib[bl, pl.ds(cc * 16, 16)]
            yi = yib[bl, pl.ds(cc * 16, 16)]
            mask = (yi // CROWS) == chunk
            vals = values(cc)
            plsc.store_scatter(buf, [yi % CROWS, xi], vals, mask=mask)

    def _emit(c, parity, first):
        bl, ch, chunk = _combo(c)
        gb = wid * B_PER_W + bl
        buf, sem = bufs[parity], sems[parity]
        if not first:
            blp, _, chunkp = _combo(c - 2)
            pltpu.make_async_copy(
                buf, out_hbm.at[gb, ch, pl.ds(chunk * CROWS, CROWS)],
                sem).wait()
            _scatter(buf, blp, chunkp,
                     lambda cc: jnp.zeros((16,), jnp.float32))
        _scatter(buf, bl, chunk,
                 lambda cc: vab[2 * bl + ch, pl.ds(cc * 16, 16)])
        pltpu.async_copy(
            buf, out_hbm.at[gb, ch, pl.ds(chunk * CROWS, CROWS)], sem)

    _emit(0, 0, True)
    _emit(1, 1, True)

    def _body(k, carry):
        _emit(2 * k, 0, False)
        _emit(2 * k + 1, 1, False)
        return carry

    lax.fori_loop(1, NCOMBO // 2, _body, 0)

    blz, chz, chkz = _combo(NCOMBO - 2)
    gbz = wid * B_PER_W + blz
    pltpu.make_async_copy(
        buf0, out_hbm.at[gbz, chz, pl.ds(chkz * CROWS, CROWS)], sem0).wait()
    blz, chz, chkz = _combo(NCOMBO - 1)
    gbz = wid * B_PER_W + blz
    pltpu.make_async_copy(
        buf1, out_hbm.at[gbz, chz, pl.ds(chkz * CROWS, CROWS)], sem1).wait()


def kernel(x_vecs, device_weights, device_bias, category_weights,
           category_bias):
    return _vec2im_sc(x_vecs.reshape(-1), device_weights, device_bias,
                      category_weights, category_bias)


# speedup vs baseline: 12.6927x; 1.0182x over previous
import functools

import jax
import jax.numpy as jnp
from jax import lax
from jax.experimental import pallas as pl
from jax.experimental.pallas import tpu as pltpu
from jax.experimental.pallas import tpu_sc as plsc

B, R, H, W = 128, 34, 512, 512
NUM_CATS = 5
NW = 32
B_PER_W = B // NW
CROWS = 64
NCHUNK = H // CROWS
NCOMBO = B_PER_W * 2 * NCHUNK
RPAD = 48

_mesh = plsc.VectorSubcoreMesh(core_axis_name="c", subcore_axis_name="s")


@functools.partial(
    pl.kernel,
    mesh=_mesh,
    compiler_params=pltpu.CompilerParams(needs_layout_passes=False),
    out_type=jax.ShapeDtypeStruct((B, 2, H, W), jnp.float32),
    scratch_types=[
        pltpu.VMEM((CROWS, W), jnp.float32),
        pltpu.VMEM((CROWS, W), jnp.float32),
        pltpu.VMEM((CROWS, W), jnp.float32),
        pltpu.VMEM((B_PER_W * R * 4,), jnp.float32),
        pltpu.VMEM((R,), jnp.float32),
        pltpu.VMEM((R,), jnp.float32),
        pltpu.VMEM((NUM_CATS,), jnp.float32),
        pltpu.VMEM((NUM_CATS,), jnp.float32),
        pltpu.VMEM((B_PER_W, RPAD), jnp.int32),
        pltpu.VMEM((B_PER_W, RPAD), jnp.int32),
        pltpu.VMEM((2 * B_PER_W, RPAD), jnp.float32),
        pltpu.SemaphoreType.DMA,
        pltpu.SemaphoreType.DMA,
        pltpu.SemaphoreType.DMA,
        pltpu.SemaphoreType.DMA,
    ],
)
def _vec2im_sc(x_hbm, dw_hbm, db_hbm, cw_hbm, cb_hbm, out_hbm,
               buf0, buf1, buf2, xvb, dwb, dbb, cwb, cbb, xib, yib, vab,
               sem0, sem1, sem2, sems_stage):
    wid = lax.axis_index("s") * 2 + lax.axis_index("c")
    bufs = (buf0, buf1, buf2)
    sems = (sem0, sem1, sem2)

    stage = [
        pltpu.async_copy(
            x_hbm.at[pl.ds(wid * (B_PER_W * R * 4), B_PER_W * R * 4)],
            xvb, sems_stage),
        pltpu.async_copy(dw_hbm, dwb, sems_stage),
        pltpu.async_copy(db_hbm, dbb, sems_stage),
        pltpu.async_copy(cw_hbm, cwb, sems_stage),
        pltpu.async_copy(cb_hbm, cbb, sems_stage),
    ]

    def _zrow(r, carry):
        z16 = jnp.zeros((16,), jnp.float32)
        for j in range(W // 16):
            buf0[r, pl.ds(j * 16, 16)] = z16
            buf1[r, pl.ds(j * 16, 16)] = z16
            buf2[r, pl.ds(j * 16, 16)] = z16
        return carry

    lax.fori_loop(0, CROWS, _zrow, 0)
    for cp in stage:
        cp.wait()

    lane = lax.iota(jnp.int32, 16)
    for bl in range(B_PER_W):
        for c in range(RPAD // 16):
            rc = jnp.minimum(lane + c * 16, R - 1)
            base = bl * (R * 4) + rc * 4
            p = plsc.load_gather(xvb, [base])
            xf = plsc.load_gather(xvb, [base + 1])
            yf = plsc.load_gather(xvb, [base + 2])
            cf = plsc.load_gather(xvb, [base + 3])
            dw = plsc.load_gather(dwb, [rc])
            db = plsc.load_gather(dbb, [rc])
            cat = cf.astype(jnp.int32)
            cw = plsc.load_gather(cwb, [cat])
            cb = plsc.load_gather(cbb, [cat])
            ind = jnp.where(p != 0.0, 1.0, 0.0)
            v0 = (p * dw + ind * db) * cw + ind * cb
            xib[bl, pl.ds(c * 16, 16)] = xf.astype(jnp.int32)
            yib[bl, pl.ds(c * 16, 16)] = yf.astype(jnp.int32)
            vab[2 * bl, pl.ds(c * 16, 16)] = v0
            vab[2 * bl + 1, pl.ds(c * 16, 16)] = p

    def _combo(c):
        return c // (2 * NCHUNK), (c // NCHUNK) % 2, c % NCHUNK

    def _scatter(buf, bl, chunk, values):
        for cc in range(RPAD // 16):
            xi = xib[bl, pl.ds(cc * 16, 16)]
            yi = yib[bl, pl.ds(cc * 16, 16)]
            mask = (yi // CROWS) == chunk
            vals = values(cc)
            plsc.store_scatter(buf, [yi % CROWS, xi], vals, mask=mask)

    def _emit(c, parity, first):
        bl, ch, chunk = _combo(c)
        gb = wid * B_PER_W + bl
        buf, sem = bufs[parity], sems[parity]
        if not first:
            blp, _, chunkp = _combo(c - 3)
            pltpu.make_async_copy(
                buf, out_hbm.at[gb, ch, pl.ds(chunk * CROWS, CROWS)],
                sem).wait()
            _scatter(buf, blp, chunkp,
                     lambda cc: jnp.zeros((16,), jnp.float32))
        _scatter(buf, bl, chunk,
                 lambda cc: vab[2 * bl + ch, pl.ds(cc * 16, 16)])
        pltpu.async_copy(
            buf, out_hbm.at[gb, ch, pl.ds(chunk * CROWS, CROWS)], sem)

    _emit(0, 0, True)
    _emit(1, 1, True)
    _emit(2, 2, True)

    def _body(k, carry):
        _emit(3 * k, 0, False)
        _emit(3 * k + 1, 1, False)
        _emit(3 * k + 2, 2, False)
        return carry

    lax.fori_loop(1, (NCOMBO - 1) // 3, _body, 0)
    _emit(NCOMBO - 1, (NCOMBO - 1) % 3, False)

    for cz in (NCOMBO - 3, NCOMBO - 2, NCOMBO - 1):
        blz, chz, chkz = _combo(cz)
        gbz = wid * B_PER_W + blz
        pltpu.make_async_copy(
            bufs[cz % 3], out_hbm.at[gbz, chz, pl.ds(chkz * CROWS, CROWS)],
            sems[cz % 3]).wait()


def kernel(x_vecs, device_weights, device_bias, category_weights,
           category_bias):
    return _vec2im_sc(x_vecs.reshape(-1), device_weights, device_bias,
                      category_weights, category_bias)
